# baseline (speedup 1.0000x reference)
import math

import jax
import jax.numpy as jnp
from jax.experimental import pallas as pl
from jax.experimental.pallas import tpu as pltpu

SMOOTH = 0.1
CONF = 1.0 - SMOOTH
SEQ = 2048
VOCAB = 32000
RB = 1024
TILE = 6400
NR = SEQ // RB
NT = VOCAB // TILE
EPS = SMOOTH / (VOCAB - 2)
KCONST = CONF * math.log(CONF) + (VOCAB - 2) * EPS * math.log(EPS)


def _tc_body(t_ref, x_ref, out_ref):
    i = pl.program_id(0)
    j = pl.program_id(1)
    t = t_ref[:, :1]
    m = (t != 0).astype(jnp.float32)
    x = x_ref[...]
    tloc = jnp.where(t == 0, -1, t - j * TILE)
    col = jax.lax.broadcasted_iota(jnp.int32, (RB, TILE), 1)
    z = jnp.where(col == tloc, x, 0.0)
    zrow = jnp.sum(z, axis=1, keepdims=True)
    s = jnp.sum(x, axis=1, keepdims=True)

    @pl.when((i == 0) & (j == 0))
    def _():
        out_ref[0, 0] = 0.0

    @pl.when(j == 0)
    def _():
        out_ref[0, 0] += KCONST * jnp.sum(m) + EPS * jnp.sum(x[:, :1] * m)

    out_ref[0, 0] += jnp.sum((EPS - CONF) * zrow - EPS * (s * m))


def kernel(x, target_sequence):
    x2 = x.reshape(SEQ, VOCAB)
    t2 = target_sequence.reshape(SEQ, 1).astype(jnp.int32)
    out = pl.pallas_call(
        _tc_body,
        grid=(NR, NT),
        in_specs=[
            pl.BlockSpec((RB, 1), lambda i, j: (i, 0)),
            pl.BlockSpec((RB, TILE), lambda i, j: (i, j)),
        ],
        out_specs=pl.BlockSpec(memory_space=pltpu.SMEM),
        out_shape=jax.ShapeDtypeStruct((1, 1), jnp.float32),
    )(t2, x2)
    return out[0, 0]


# speedup vs baseline: 1.0134x; 1.0134x over previous
import math

import jax
import jax.numpy as jnp
from jax.experimental import pallas as pl
from jax.experimental.pallas import tpu as pltpu

SMOOTH = 0.1
CONF = 1.0 - SMOOTH
SEQ = 2048
VOCAB = 32000
RB = 1024
TILE = 3200
NR = SEQ // RB
NT = VOCAB // TILE
EPS = SMOOTH / (VOCAB - 2)
KCONST = CONF * math.log(CONF) + (VOCAB - 2) * EPS * math.log(EPS)


def _tc_body(t_ref, x_ref, out_ref):
    i = pl.program_id(0)
    j = pl.program_id(1)
    t = t_ref[:, :1]
    m = (t != 0).astype(jnp.float32)
    x = x_ref[...]
    tloc = jnp.where(t == 0, -1, t - j * TILE)
    col = jax.lax.broadcasted_iota(jnp.int32, (RB, TILE), 1)
    z = jnp.where(col == tloc, x, 0.0)
    zrow = jnp.sum(z, axis=1, keepdims=True)
    s = jnp.sum(x, axis=1, keepdims=True)

    @pl.when((i == 0) & (j == 0))
    def _():
        out_ref[0, 0] = 0.0

    @pl.when(j == 0)
    def _():
        out_ref[0, 0] += KCONST * jnp.sum(m) + EPS * jnp.sum(x[:, :1] * m)

    out_ref[0, 0] += jnp.sum((EPS - CONF) * zrow - EPS * (s * m))


def kernel(x, target_sequence):
    x2 = x.reshape(SEQ, VOCAB)
    t2 = target_sequence.reshape(SEQ, 1).astype(jnp.int32)
    out = pl.pallas_call(
        _tc_body,
        grid=(NR, NT),
        in_specs=[
            pl.BlockSpec((RB, 1), lambda i, j: (i, 0)),
            pl.BlockSpec((RB, TILE), lambda i, j: (i, j)),
        ],
        out_specs=pl.BlockSpec(memory_space=pltpu.SMEM),
        out_shape=jax.ShapeDtypeStruct((1, 1), jnp.float32),
    )(t2, x2)
    return out[0, 0]
